# baseline (speedup 1.0000x reference)
import jax
import jax.numpy as jnp
from jax import lax
from jax.experimental import pallas as pl
from jax.experimental.pallas import tpu as pltpu
from jax.experimental.pallas import tpu_sc as plsc

LANES = 16
CHUNK = 80
NC = 2
NS = 16
NW = NC * NS


def _lo_f32(v):
    return lax.bitcast_convert_type(lax.shift_left(v, 16), jnp.float32)


def _hi_f32(v):
    return lax.bitcast_convert_type(
        lax.bitwise_and(v, jnp.int32(-65536)), jnp.float32)


def _hadd(a, b, eidx, oidx, lo_mask):
    sa = jnp.take_along_axis(a, eidx, axis=0) + jnp.take_along_axis(a, oidx, axis=0)
    sb = jnp.take_along_axis(b, eidx, axis=0) + jnp.take_along_axis(b, oidx, axis=0)
    return jnp.where(lo_mask, sa, sb)


def _sc_body(emb_hbm, nx_hbm, ny_hbm, out_hbm,
             idx_x0, idx_x1, idx_y0, idx_y1,
             rows_x0, rows_x1, rows_y0, rows_y1,
             scores0, scores1,
             s_ix0, s_ix1, s_iy0, s_iy1, s_gx0, s_gx1, s_gy0, s_gy1,
             s_o0, s_o1):
    idx_x = (idx_x0, idx_x1)
    idx_y = (idx_y0, idx_y1)
    rows_x = (rows_x0, rows_x1)
    rows_y = (rows_y0, rows_y1)
    scores = (scores0, scores1)
    s_ix = (s_ix0, s_ix1)
    s_iy = (s_iy0, s_iy1)
    s_gx = (s_gx0, s_gx1)
    s_gy = (s_gy0, s_gy1)
    s_o = (s_o0, s_o1)

    dim_words = emb_hbm.shape[1]
    n_pairs = nx_hbm.shape[0]
    n_chunks = n_pairs // CHUNK
    w = lax.axis_index("s") * NC + lax.axis_index("c")
    n = (n_chunks - w + NW - 1) // NW

    iota = lax.iota(jnp.int32, LANES)
    eidx = (2 * iota) % LANES
    oidx = (2 * iota + 1) % LANES
    lo_mask = iota < (LANES // 2)

    def base(c):
        return (w + c * NW) * CHUNK

    def fire_idx(c, b):
        pltpu.async_copy(nx_hbm.at[pl.ds(base(c), CHUNK)], idx_x[b], s_ix[b])
        pltpu.async_copy(ny_hbm.at[pl.ds(base(c), CHUNK)], idx_y[b], s_iy[b])

    def wait_idx(c, b):
        pltpu.make_async_copy(
            nx_hbm.at[pl.ds(base(c), CHUNK)], idx_x[b], s_ix[b]).wait()
        pltpu.make_async_copy(
            ny_hbm.at[pl.ds(base(c), CHUNK)], idx_y[b], s_iy[b]).wait()

    def fire_gather(b):
        pltpu.async_copy(emb_hbm.at[idx_x[b]], rows_x[b], s_gx[b])
        pltpu.async_copy(emb_hbm.at[idx_y[b]], rows_y[b], s_gy[b])

    def wait_gather(b):
        pltpu.make_async_copy(emb_hbm.at[idx_x[b]], rows_x[b], s_gx[b]).wait()
        pltpu.make_async_copy(emb_hbm.at[idx_y[b]], rows_y[b], s_gy[b]).wait()

    def fire_out(c, b):
        pltpu.async_copy(scores[b], out_hbm.at[pl.ds(base(c), CHUNK)], s_o[b])

    def wait_out(c, b):
        pltpu.make_async_copy(
            scores[b], out_hbm.at[pl.ds(base(c), CHUNK)], s_o[b]).wait()

    def compute(c, b):
        rx = rows_x[b]
        ry = rows_y[b]

        @plsc.parallel_loop(0, CHUNK // LANES)
        def _group(g):
            accs = []
            for l in range(LANES):
                p = g * LANES + l
                vx = rx[p, pl.ds(0, LANES)]
                vy = ry[p, pl.ds(0, LANES)]
                acc_e = _lo_f32(vx) * _lo_f32(vy)
                acc_o = _hi_f32(vx) * _hi_f32(vy)
                for j in range(1, dim_words // LANES):
                    vx = rx[p, pl.ds(j * LANES, LANES)]
                    vy = ry[p, pl.ds(j * LANES, LANES)]
                    acc_e = acc_e + _lo_f32(vx) * _lo_f32(vy)
                    acc_o = acc_o + _hi_f32(vx) * _hi_f32(vy)
                accs.append(acc_e + acc_o)
            while len(accs) > 1:
                accs = [_hadd(accs[2 * i], accs[2 * i + 1], eidx, oidx, lo_mask)
                        for i in range(len(accs) // 2)]
            scores[b][pl.ds(g * LANES, LANES)] = accs[0]

    fire_idx(0, 0)
    fire_idx(1, 1)
    wait_idx(0, 0)
    fire_gather(0)

    n_loop = (n_chunks + NW - 1) // NW

    @pl.loop(0, n_loop, step=2)
    def _outer(c0):
        for b in range(2):
            c = c0 + b
            q = 1 - b

            plsc.subcore_barrier()

            @pl.when(c < n)
            def _():
                wait_gather(b)

                @pl.when(c + 1 < n)
                def _():
                    wait_idx(c + 1, q)
                    fire_gather(q)

                @pl.when(c + 2 < n)
                def _():
                    fire_idx(c + 2, b)

                @pl.when(c >= 2)
                def _():
                    wait_out(c - 2, b)

                compute(c, b)
                fire_out(c, b)

    @pl.when(n % 2 == 1)
    def _():
        wait_out(n - 2, 1)
        wait_out(n - 1, 0)

    @pl.when(n % 2 == 0)
    def _():
        wait_out(n - 2, 0)
        wait_out(n - 1, 1)


def kernel(embeddings, nodes_x, nodes_y):
    n_pairs = nodes_x.shape[0]
    n_nodes, dim = embeddings.shape
    emb_bf = embeddings.astype(jnp.bfloat16)
    emb_packed = lax.bitcast_convert_type(
        emb_bf.reshape(n_nodes, dim // 2, 2), jnp.int32)
    dim_words = dim // 2
    mesh = plsc.VectorSubcoreMesh(core_axis_name="c", subcore_axis_name="s")
    k = pl.kernel(
        _sc_body,
        out_type=jax.ShapeDtypeStruct((n_pairs,), jnp.float32),
        mesh=mesh,
        scratch_types=[
            pltpu.VMEM((CHUNK,), jnp.int32),
            pltpu.VMEM((CHUNK,), jnp.int32),
            pltpu.VMEM((CHUNK,), jnp.int32),
            pltpu.VMEM((CHUNK,), jnp.int32),
            pltpu.VMEM((CHUNK, dim_words), jnp.int32),
            pltpu.VMEM((CHUNK, dim_words), jnp.int32),
            pltpu.VMEM((CHUNK, dim_words), jnp.int32),
            pltpu.VMEM((CHUNK, dim_words), jnp.int32),
            pltpu.VMEM((CHUNK,), jnp.float32),
            pltpu.VMEM((CHUNK,), jnp.float32),
            pltpu.SemaphoreType.DMA,
            pltpu.SemaphoreType.DMA,
            pltpu.SemaphoreType.DMA,
            pltpu.SemaphoreType.DMA,
            pltpu.SemaphoreType.DMA,
            pltpu.SemaphoreType.DMA,
            pltpu.SemaphoreType.DMA,
            pltpu.SemaphoreType.DMA,
            pltpu.SemaphoreType.DMA,
            pltpu.SemaphoreType.DMA,
        ],
    )
    return k(emb_packed, nodes_x.astype(jnp.int32), nodes_y.astype(jnp.int32))


# speedup vs baseline: 1.0737x; 1.0737x over previous
import jax
import jax.numpy as jnp
from jax import lax
from jax.experimental import pallas as pl
from jax.experimental.pallas import tpu as pltpu
from jax.experimental.pallas import tpu_sc as plsc

LANES = 16
CHUNK = 128
NC = 2
NS = 16
NW = NC * NS


def _lo_f32(v):
    return lax.bitcast_convert_type(lax.shift_left(v, 16), jnp.float32)


def _hi_f32(v):
    return lax.bitcast_convert_type(
        lax.bitwise_and(v, jnp.int32(-65536)), jnp.float32)


def _hadd(a, b, eidx, oidx, lo_mask):
    sa = jnp.take_along_axis(a, eidx, axis=0) + jnp.take_along_axis(a, oidx, axis=0)
    sb = jnp.take_along_axis(b, eidx, axis=0) + jnp.take_along_axis(b, oidx, axis=0)
    return jnp.where(lo_mask, sa, sb)


def _sc_body(emb_hbm, nx_hbm, ny_hbm, out_hbm,
             idx_x0, idx_x1, idx_y0, idx_y1,
             rows_x0, rows_x1, rows_y0, rows_y1,
             scores0, scores1,
             s_ix0, s_ix1, s_iy0, s_iy1, s_gx0, s_gx1, s_gy0, s_gy1,
             s_o0, s_o1):
    idx_x = (idx_x0, idx_x1)
    idx_y = (idx_y0, idx_y1)
    rows_x = (rows_x0, rows_x1)
    rows_y = (rows_y0, rows_y1)
    scores = (scores0, scores1)
    s_ix = (s_ix0, s_ix1)
    s_iy = (s_iy0, s_iy1)
    s_gx = (s_gx0, s_gx1)
    s_gy = (s_gy0, s_gy1)
    s_o = (s_o0, s_o1)

    dim_words = emb_hbm.shape[1]
    n_pairs = nx_hbm.shape[0]
    n_chunks = n_pairs // CHUNK
    w = lax.axis_index("s") * NC + lax.axis_index("c")
    n = (n_chunks - w + NW - 1) // NW

    iota = lax.iota(jnp.int32, LANES)
    eidx = (2 * iota) % LANES
    oidx = (2 * iota + 1) % LANES
    lo_mask = iota < (LANES // 2)

    def base(c):
        return (w + c * NW) * CHUNK

    def fire_idx(c, b):
        pltpu.async_copy(nx_hbm.at[pl.ds(base(c), CHUNK)], idx_x[b], s_ix[b])
        pltpu.async_copy(ny_hbm.at[pl.ds(base(c), CHUNK)], idx_y[b], s_iy[b])

    def wait_idx(c, b):
        pltpu.make_async_copy(
            nx_hbm.at[pl.ds(base(c), CHUNK)], idx_x[b], s_ix[b]).wait()
        pltpu.make_async_copy(
            ny_hbm.at[pl.ds(base(c), CHUNK)], idx_y[b], s_iy[b]).wait()

    def fire_gather(b):
        pltpu.async_copy(emb_hbm.at[idx_x[b]], rows_x[b], s_gx[b])
        pltpu.async_copy(emb_hbm.at[idx_y[b]], rows_y[b], s_gy[b])

    def wait_gather(b):
        pltpu.make_async_copy(emb_hbm.at[idx_x[b]], rows_x[b], s_gx[b]).wait()
        pltpu.make_async_copy(emb_hbm.at[idx_y[b]], rows_y[b], s_gy[b]).wait()

    def fire_out(c, b):
        pltpu.async_copy(scores[b], out_hbm.at[pl.ds(base(c), CHUNK)], s_o[b])

    def wait_out(c, b):
        pltpu.make_async_copy(
            scores[b], out_hbm.at[pl.ds(base(c), CHUNK)], s_o[b]).wait()

    def compute(c, b):
        rx = rows_x[b]
        ry = rows_y[b]

        @plsc.parallel_loop(0, CHUNK // LANES)
        def _group(g):
            accs = []
            for l in range(LANES):
                p = g * LANES + l
                acc = [None] * 4
                for j in range(dim_words // LANES):
                    vx = rx[p, pl.ds(j * LANES, LANES)]
                    vy = ry[p, pl.ds(j * LANES, LANES)]
                    pe = _lo_f32(vx) * _lo_f32(vy)
                    po = _hi_f32(vx) * _hi_f32(vy)
                    ke = 2 * (j % 2)
                    acc[ke] = pe if acc[ke] is None else acc[ke] + pe
                    acc[ke + 1] = po if acc[ke + 1] is None else acc[ke + 1] + po
                accs.append((acc[0] + acc[1]) + (acc[2] + acc[3]))
            while len(accs) > 1:
                accs = [_hadd(accs[2 * i], accs[2 * i + 1], eidx, oidx, lo_mask)
                        for i in range(len(accs) // 2)]
            scores[b][pl.ds(g * LANES, LANES)] = accs[0]

    fire_idx(0, 0)
    fire_idx(1, 1)
    wait_idx(0, 0)
    fire_gather(0)

    n_loop = (n_chunks + NW - 1) // NW

    @pl.loop(0, n_loop, step=2)
    def _outer(c0):
        for b in range(2):
            c = c0 + b
            q = 1 - b

            @pl.when(c < n)
            def _():
                wait_gather(b)

                @pl.when(c + 1 < n)
                def _():
                    wait_idx(c + 1, q)
                    fire_gather(q)

                @pl.when(c + 2 < n)
                def _():
                    fire_idx(c + 2, b)

                @pl.when(c >= 2)
                def _():
                    wait_out(c - 2, b)

                compute(c, b)
                fire_out(c, b)

    @pl.when(n % 2 == 1)
    def _():
        wait_out(n - 2, 1)
        wait_out(n - 1, 0)

    @pl.when(n % 2 == 0)
    def _():
        wait_out(n - 2, 0)
        wait_out(n - 1, 1)


def kernel(embeddings, nodes_x, nodes_y):
    n_pairs = nodes_x.shape[0]
    n_nodes, dim = embeddings.shape
    emb_bf = embeddings.astype(jnp.bfloat16)
    emb_packed = lax.bitcast_convert_type(
        emb_bf.reshape(n_nodes, dim // 2, 2), jnp.int32)
    dim_words = dim // 2
    mesh = plsc.VectorSubcoreMesh(core_axis_name="c", subcore_axis_name="s")
    k = pl.kernel(
        _sc_body,
        out_type=jax.ShapeDtypeStruct((n_pairs,), jnp.float32),
        mesh=mesh,
        scratch_types=[
            pltpu.VMEM((CHUNK,), jnp.int32),
            pltpu.VMEM((CHUNK,), jnp.int32),
            pltpu.VMEM((CHUNK,), jnp.int32),
            pltpu.VMEM((CHUNK,), jnp.int32),
            pltpu.VMEM((CHUNK, dim_words), jnp.int32),
            pltpu.VMEM((CHUNK, dim_words), jnp.int32),
            pltpu.VMEM((CHUNK, dim_words), jnp.int32),
            pltpu.VMEM((CHUNK, dim_words), jnp.int32),
            pltpu.VMEM((CHUNK,), jnp.float32),
            pltpu.VMEM((CHUNK,), jnp.float32),
            pltpu.SemaphoreType.DMA,
            pltpu.SemaphoreType.DMA,
            pltpu.SemaphoreType.DMA,
            pltpu.SemaphoreType.DMA,
            pltpu.SemaphoreType.DMA,
            pltpu.SemaphoreType.DMA,
            pltpu.SemaphoreType.DMA,
            pltpu.SemaphoreType.DMA,
            pltpu.SemaphoreType.DMA,
            pltpu.SemaphoreType.DMA,
        ],
    )
    return k(emb_packed, nodes_x.astype(jnp.int32), nodes_y.astype(jnp.int32))


# speedup vs baseline: 1.0743x; 1.0006x over previous
import jax
import jax.numpy as jnp
from jax import lax
from jax.experimental import pallas as pl
from jax.experimental.pallas import tpu as pltpu
from jax.experimental.pallas import tpu_sc as plsc

LANES = 16
CHUNK = 128
NC = 2
NS = 16
NW = NC * NS


def _lo_f32(v):
    return lax.bitcast_convert_type(lax.shift_left(v, 16), jnp.float32)


def _hi_f32(v):
    return lax.bitcast_convert_type(
        lax.bitwise_and(v, jnp.int32(-65536)), jnp.float32)


def _hadd(a, b, eidx, oidx, lo_mask):
    sa = jnp.take_along_axis(a, eidx, axis=0) + jnp.take_along_axis(a, oidx, axis=0)
    sb = jnp.take_along_axis(b, eidx, axis=0) + jnp.take_along_axis(b, oidx, axis=0)
    return jnp.where(lo_mask, sa, sb)


def _sc_body(emb_hbm, nx_hbm, ny_hbm, out_hbm,
             idx_x0, idx_x1, idx_y0, idx_y1,
             rows_x0, rows_x1, rows_y0, rows_y1,
             scores0, scores1,
             s_ix0, s_ix1, s_iy0, s_iy1, s_gx0, s_gx1, s_gy0, s_gy1,
             s_o0, s_o1):
    idx_x = (idx_x0, idx_x1)
    idx_y = (idx_y0, idx_y1)
    rows_x = (rows_x0, rows_x1)
    rows_y = (rows_y0, rows_y1)
    scores = (scores0, scores1)
    s_ix = (s_ix0, s_ix1)
    s_iy = (s_iy0, s_iy1)
    s_gx = (s_gx0, s_gx1)
    s_gy = (s_gy0, s_gy1)
    s_o = (s_o0, s_o1)

    dim_words = emb_hbm.shape[1]
    n_pairs = nx_hbm.shape[0]
    n_chunks = n_pairs // CHUNK
    w = lax.axis_index("s") * NC + lax.axis_index("c")
    n = (n_chunks - w + NW - 1) // NW

    iota = lax.iota(jnp.int32, LANES)
    eidx = (2 * iota) % LANES
    oidx = (2 * iota + 1) % LANES
    lo_mask = iota < (LANES // 2)

    def base(c):
        return (w + c * NW) * CHUNK

    def fire_idx(c, b):
        pltpu.async_copy(nx_hbm.at[pl.ds(base(c), CHUNK)], idx_x[b], s_ix[b])
        pltpu.async_copy(ny_hbm.at[pl.ds(base(c), CHUNK)], idx_y[b], s_iy[b])

    def wait_idx(c, b):
        pltpu.make_async_copy(
            nx_hbm.at[pl.ds(base(c), CHUNK)], idx_x[b], s_ix[b]).wait()
        pltpu.make_async_copy(
            ny_hbm.at[pl.ds(base(c), CHUNK)], idx_y[b], s_iy[b]).wait()

    def fire_gather(b):
        pltpu.async_copy(emb_hbm.at[idx_x[b]], rows_x[b], s_gx[b])
        pltpu.async_copy(emb_hbm.at[idx_y[b]], rows_y[b], s_gy[b])

    def wait_gather(b):
        pltpu.make_async_copy(emb_hbm.at[idx_x[b]], rows_x[b], s_gx[b]).wait()
        pltpu.make_async_copy(emb_hbm.at[idx_y[b]], rows_y[b], s_gy[b]).wait()

    def fire_out(c, b):
        pltpu.async_copy(scores[b], out_hbm.at[pl.ds(base(c), CHUNK)], s_o[b])

    def wait_out(c, b):
        pltpu.make_async_copy(
            scores[b], out_hbm.at[pl.ds(base(c), CHUNK)], s_o[b]).wait()

    def compute(c, b):
        rx = rows_x[b]
        ry = rows_y[b]

        @plsc.parallel_loop(0, CHUNK // LANES)
        def _group(g):
            def _init(l):
                p = g * LANES + l
                vx = rx[p, pl.ds(0, LANES)]
                vy = ry[p, pl.ds(0, LANES)]
                return _lo_f32(vx) * _lo_f32(vy) + _hi_f32(vx) * _hi_f32(vy)

            @pl.loop(1, dim_words // LANES,
                     init_carry=tuple(_init(l) for l in range(LANES)))
            def _jstep(j, carry):
                o = pl.multiple_of(j * LANES, LANES)
                new = []
                for l in range(LANES):
                    p = g * LANES + l
                    vx = rx[p, pl.ds(o, LANES)]
                    vy = ry[p, pl.ds(o, LANES)]
                    new.append(carry[l]
                               + (_lo_f32(vx) * _lo_f32(vy)
                                  + _hi_f32(vx) * _hi_f32(vy)))
                return tuple(new)

            accs = list(_jstep)
            while len(accs) > 1:
                accs = [_hadd(accs[2 * i], accs[2 * i + 1], eidx, oidx, lo_mask)
                        for i in range(len(accs) // 2)]
            scores[b][pl.ds(g * LANES, LANES)] = accs[0]

    fire_idx(0, 0)
    fire_idx(1, 1)
    wait_idx(0, 0)
    fire_gather(0)

    n_loop = (n_chunks + NW - 1) // NW

    @pl.loop(0, n_loop, step=2)
    def _outer(c0):
        for b in range(2):
            c = c0 + b
            q = 1 - b

            @pl.when(c < n)
            def _():
                wait_gather(b)

                @pl.when(c + 1 < n)
                def _():
                    wait_idx(c + 1, q)
                    fire_gather(q)

                @pl.when(c + 2 < n)
                def _():
                    fire_idx(c + 2, b)

                @pl.when(c >= 2)
                def _():
                    wait_out(c - 2, b)

                compute(c, b)
                fire_out(c, b)

    @pl.when(n % 2 == 1)
    def _():
        wait_out(n - 2, 1)
        wait_out(n - 1, 0)

    @pl.when(n % 2 == 0)
    def _():
        wait_out(n - 2, 0)
        wait_out(n - 1, 1)


def kernel(embeddings, nodes_x, nodes_y):
    n_pairs = nodes_x.shape[0]
    n_nodes, dim = embeddings.shape
    emb_bf = embeddings.astype(jnp.bfloat16)
    emb_packed = lax.bitcast_convert_type(
        emb_bf.reshape(n_nodes, dim // 2, 2), jnp.int32)
    dim_words = dim // 2
    mesh = plsc.VectorSubcoreMesh(core_axis_name="c", subcore_axis_name="s")
    k = pl.kernel(
        _sc_body,
        out_type=jax.ShapeDtypeStruct((n_pairs,), jnp.float32),
        mesh=mesh,
        scratch_types=[
            pltpu.VMEM((CHUNK,), jnp.int32),
            pltpu.VMEM((CHUNK,), jnp.int32),
            pltpu.VMEM((CHUNK,), jnp.int32),
            pltpu.VMEM((CHUNK,), jnp.int32),
            pltpu.VMEM((CHUNK, dim_words), jnp.int32),
            pltpu.VMEM((CHUNK, dim_words), jnp.int32),
            pltpu.VMEM((CHUNK, dim_words), jnp.int32),
            pltpu.VMEM((CHUNK, dim_words), jnp.int32),
            pltpu.VMEM((CHUNK,), jnp.float32),
            pltpu.VMEM((CHUNK,), jnp.float32),
            pltpu.SemaphoreType.DMA,
            pltpu.SemaphoreType.DMA,
            pltpu.SemaphoreType.DMA,
            pltpu.SemaphoreType.DMA,
            pltpu.SemaphoreType.DMA,
            pltpu.SemaphoreType.DMA,
            pltpu.SemaphoreType.DMA,
            pltpu.SemaphoreType.DMA,
            pltpu.SemaphoreType.DMA,
            pltpu.SemaphoreType.DMA,
        ],
    )
    return k(emb_packed, nodes_x.astype(jnp.int32), nodes_y.astype(jnp.int32))
